# baseline (speedup 1.0000x reference)
import dataclasses
import functools

import jax
import jax.numpy as jnp
import numpy as np
from jax import lax
from jax.experimental import pallas as pl
from jax.experimental.pallas import tpu as pltpu
from jax.experimental.pallas import tpu_sc as plsc

N_NODES = 10000
FEAT = 128
N_EDGES = 320000

NC = 2
NS = 16
NW = NC * NS
K = 128
NCHUNK = 78
TAIL_E = 16
NTOT = NCHUNK + 1
EPW = K * NTOT
E_PAD = NW * EPW
RPT = 624
TAIL = N_NODES - NS * RPT

_PERM = np.zeros(FEAT, np.int32)
for _q in range(FEAT // 32):
    for _c in range(32):
        _PERM[32 * _q + _c] = 32 * _q + (
            _c // 2 if _c % 2 == 0 else 16 + (_c - 1) // 2)


def _matmul(x, W):
    def body(x_ref, w_ref, o_ref):
        o_ref[...] = jax.lax.dot_general(
            x_ref[...], w_ref[...], (((1,), (0,)), ((), ())),
            preferred_element_type=jnp.float32,
            precision=jax.lax.Precision.HIGHEST)

    return pl.pallas_call(
        body,
        out_shape=jax.ShapeDtypeStruct((N_NODES, FEAT), jnp.float32),
    )(x, W)


def _combine(parts):
    def body(p_ref, o_ref):
        o_ref[...] = p_ref[0] + p_ref[1]

    return pl.pallas_call(
        body,
        out_shape=jax.ShapeDtypeStruct((N_NODES, FEAT), jnp.float32),
    )(parts)


def _scale_rows(rows_v, meta_v, scaled_v):
    @pl.loop(0, K, step=16)
    def _(g):
        wv = plsc.bitcast(meta_v[2, pl.ds(g, 16)], jnp.float32)
        dnums = lax.GatherDimensionNumbers(
            offset_dims=(), collapsed_slice_dims=(0,), start_index_map=(0,))
        for el in range(16):
            sp = lax.gather(wv, jnp.full((16, 1), el, jnp.int32), dnums, (1,),
                            mode=lax.GatherScatterMode.PROMISE_IN_BOUNDS)
            e = g + el
            for q in range(4):
                v = rows_v[e, pl.ds(q * 16, 16)]
                vb = plsc.bitcast(v, jnp.bfloat16)
                a, b = plsc.unpack(vb, format=plsc.PackFormat.INTERLEAVED)
                scaled_v[e, pl.ds(q * 32, 16)] = a * sp
                scaled_v[e, pl.ds(q * 32 + 16, 16)] = b * sp


def _sc_aggregate(hp, meta, zeros):
    mesh = plsc.VectorSubcoreMesh(core_axis_name="c", subcore_axis_name="s",
                                  num_cores=NC, num_subcores=NS)
    cp = pltpu.CompilerParams()
    if "needs_layout_passes" in pltpu.CompilerParams.__dataclass_fields__:
        cp = dataclasses.replace(cp, needs_layout_passes=False)
    cp = dataclasses.replace(cp, use_tc_tiling_on_sc=False)

    @functools.partial(
        pl.kernel,
        out_type=jax.ShapeDtypeStruct((NC, N_NODES, FEAT), jnp.float32),
        mesh=mesh,
        scratch_types=[
            pltpu.VMEM((K, FEAT // 2), jnp.int32),
            pltpu.VMEM((K, FEAT // 2), jnp.int32),
            pltpu.VMEM((4, K), jnp.int32),
            pltpu.VMEM((4, K), jnp.int32),
            pltpu.VMEM((K, FEAT), jnp.float32),
            pltpu.VMEM((K, FEAT), jnp.float32),
            pltpu.VMEM((K,), jnp.int32),
            pltpu.VMEM((K,), jnp.int32),
            pltpu.VMEM((TAIL_E,), jnp.int32),
            pltpu.VMEM_SHARED((N_NODES, FEAT), jnp.float32),
            pltpu.SemaphoreType.DMA,
            pltpu.SemaphoreType.DMA,
            pltpu.SemaphoreType.DMA,
            pltpu.SemaphoreType.DMA,
            pltpu.SemaphoreType.DMA,
            pltpu.SemaphoreType.DMA,
        ],
        compiler_params=cp,
    )
    def k(h_hbm, meta_hbm, z_hbm, out_hbm,
          rows0, rows1, meta0, meta1, scaled0, scaled1, dstb0, dstb1, dstt,
          acc_s,
          gsem0, gsem1, msem0, msem1, ssem0, ssem1):
        cid = lax.axis_index("c")
        sid = lax.axis_index("s")
        wid = cid * NS + sid

        r0 = sid * RPT
        pltpu.sync_copy(z_hbm.at[pl.ds(r0, RPT)], acc_s.at[pl.ds(r0, RPT)])

        @pl.when(sid == NS - 1)
        def _():
            pltpu.sync_copy(z_hbm.at[pl.ds(NS * RPT, TAIL)],
                            acc_s.at[pl.ds(NS * RPT, TAIL)])

        plsc.subcore_barrier()

        pltpu.sync_copy(meta_hbm.at[wid, 0, pl.ds(0, 4)], meta0)
        pltpu.async_copy(h_hbm.at[meta0.at[0]], rows0, gsem0)
        pltpu.async_copy(meta_hbm.at[wid, 1, pl.ds(0, 4)], meta1, msem1)

        def chunk(kk, rows_c, meta_c, gsem_c, msem_c, scaled_c, dstb_c,
                  ssem_c, rows_n, meta_n, gsem_n, msem_n):
            @pl.when(kk + 1 < NCHUNK)
            def _():
                pltpu.make_async_copy(
                    meta_hbm.at[wid, kk + 1, pl.ds(0, 4)],
                    meta_n, msem_n).wait()
                pltpu.async_copy(h_hbm.at[meta_n.at[0]], rows_n, gsem_n)

            pltpu.make_async_copy(h_hbm.at[meta_c.at[0]], rows_c, gsem_c).wait()

            @pl.when(kk >= 2)
            def _():
                pltpu.make_async_copy(scaled_c, acc_s.at[dstb_c], ssem_c).wait()

            _scale_rows(rows_c, meta_c, scaled_c)

            for j in range(8):
                sl = pl.ds(j * 16, 16)
                dstb_c[sl] = meta_c[1, sl]
            pltpu.async_copy(scaled_c, acc_s.at[dstb_c], ssem_c, add=True)

            @pl.when(kk + 2 < NCHUNK)
            def _():
                pltpu.async_copy(meta_hbm.at[wid, kk + 2, pl.ds(0, 4)],
                                 meta_c, msem_c)

        @pl.loop(0, NCHUNK, step=2)
        def _(t):
            chunk(t, rows0, meta0, gsem0, msem0, scaled0, dstb0, ssem0,
                  rows1, meta1, gsem1, msem1)
            chunk(t + 1, rows1, meta1, gsem1, msem1, scaled1, dstb1, ssem1,
                  rows0, meta0, gsem0, msem0)

        pltpu.sync_copy(meta_hbm.at[wid, NCHUNK, pl.ds(0, 4)], meta0)
        pltpu.sync_copy(h_hbm.at[meta0.at[0].at[pl.ds(0, TAIL_E)]],
                        rows0.at[pl.ds(0, TAIL_E)])
        pltpu.make_async_copy(scaled0, acc_s.at[dstb0], ssem0).wait()
        wv = plsc.bitcast(meta0[2, pl.ds(0, 16)], jnp.float32)
        dnums = lax.GatherDimensionNumbers(
            offset_dims=(), collapsed_slice_dims=(0,), start_index_map=(0,))
        for el in range(TAIL_E):
            sp = lax.gather(wv, jnp.full((16, 1), el, jnp.int32), dnums, (1,),
                            mode=lax.GatherScatterMode.PROMISE_IN_BOUNDS)
            for q in range(4):
                v = rows0[el, pl.ds(q * 16, 16)]
                vb = plsc.bitcast(v, jnp.bfloat16)
                a, b = plsc.unpack(vb, format=plsc.PackFormat.INTERLEAVED)
                scaled0[el, pl.ds(q * 32, 16)] = a * sp
                scaled0[el, pl.ds(q * 32 + 16, 16)] = b * sp
        dstt[...] = meta0[1, pl.ds(0, TAIL_E)]
        pltpu.async_copy(scaled0.at[pl.ds(0, TAIL_E)], acc_s.at[dstt],
                         ssem0, add=True)

        pltpu.make_async_copy(scaled1, acc_s.at[dstb1], ssem1).wait()
        pltpu.make_async_copy(scaled0.at[pl.ds(0, TAIL_E)], acc_s.at[dstt],
                              ssem0).wait()

        plsc.subcore_barrier()
        pltpu.sync_copy(acc_s.at[pl.ds(r0, RPT)],
                        out_hbm.at[cid].at[pl.ds(r0, RPT)])

        @pl.when(sid == NS - 1)
        def _():
            pltpu.sync_copy(acc_s.at[pl.ds(NS * RPT, TAIL)],
                            out_hbm.at[cid].at[pl.ds(NS * RPT, TAIL)])

    return k(hp, meta, zeros)


def kernel(x, W, edge_index, edge_weight):
    rpw = N_EDGES // NW
    wpad = ((0, 0), (0, EPW - rpw))
    src = jnp.pad(edge_index[0].astype(jnp.int32).reshape(NW, rpw), wpad)
    dst = jnp.pad(edge_index[1].astype(jnp.int32).reshape(NW, rpw), wpad)
    ewb = jnp.pad(edge_weight.astype(jnp.float32).reshape(NW, rpw),
                  wpad).view(jnp.int32)
    fill = jnp.zeros((NW, NTOT, 1, K), jnp.int32)
    meta = jnp.concatenate(
        [src.reshape(NW, NTOT, 1, K), dst.reshape(NW, NTOT, 1, K),
         ewb.reshape(NW, NTOT, 1, K), fill], axis=2)
    Wp = jnp.take(W, jnp.asarray(_PERM), axis=1)
    h16 = _matmul(x, Wp).astype(jnp.bfloat16)
    hp = lax.bitcast_convert_type(
        h16.reshape(N_NODES, FEAT // 2, 2), jnp.int32)
    zeros = jnp.zeros((N_NODES, FEAT), jnp.float32)
    parts = _sc_aggregate(hp, meta, zeros)
    return _combine(parts)


# speedup vs baseline: 1.0003x; 1.0003x over previous
import dataclasses
import functools

import jax
import jax.numpy as jnp
import numpy as np
from jax import lax
from jax.experimental import pallas as pl
from jax.experimental.pallas import tpu as pltpu
from jax.experimental.pallas import tpu_sc as plsc

N_NODES = 10000
FEAT = 128
N_EDGES = 320000

NC = 2
NS = 16
NW = NC * NS
K = 128
NCHUNK = 80
EPW = K * NCHUNK
E_PAD = NW * EPW
RPT = 624
TAIL = N_NODES - NS * RPT

_PERM = np.zeros(FEAT, np.int32)
for _q in range(FEAT // 32):
    for _c in range(32):
        _PERM[32 * _q + _c] = 32 * _q + (
            _c // 2 if _c % 2 == 0 else 16 + (_c - 1) // 2)


def _matmul(x, W):
    def body(x_ref, w_ref, o_ref):
        o_ref[...] = jax.lax.dot_general(
            x_ref[...], w_ref[...], (((1,), (0,)), ((), ())),
            preferred_element_type=jnp.float32,
            precision=jax.lax.Precision.HIGHEST)

    return pl.pallas_call(
        body,
        out_shape=jax.ShapeDtypeStruct((N_NODES, FEAT), jnp.float32),
    )(x, W)


def _combine(parts):
    def body(p_ref, o_ref):
        o_ref[...] = p_ref[0] + p_ref[1]

    return pl.pallas_call(
        body,
        out_shape=jax.ShapeDtypeStruct((N_NODES, FEAT), jnp.float32),
    )(parts)


def _scale_rows(rows_v, meta_v, scaled_v):
    @pl.loop(0, K, step=16)
    def _(g):
        wv = plsc.bitcast(meta_v[2, pl.ds(g, 16)], jnp.float32)
        dnums = lax.GatherDimensionNumbers(
            offset_dims=(), collapsed_slice_dims=(0,), start_index_map=(0,))
        for el in range(16):
            sp = lax.gather(wv, jnp.full((16, 1), el, jnp.int32), dnums, (1,),
                            mode=lax.GatherScatterMode.PROMISE_IN_BOUNDS)
            e = g + el
            for q in range(4):
                v = rows_v[e, pl.ds(q * 16, 16)]
                vb = plsc.bitcast(v, jnp.bfloat16)
                a, b = plsc.unpack(vb, format=plsc.PackFormat.INTERLEAVED)
                scaled_v[e, pl.ds(q * 32, 16)] = a * sp
                scaled_v[e, pl.ds(q * 32 + 16, 16)] = b * sp


def _sc_aggregate(hp, meta, zeros):
    mesh = plsc.VectorSubcoreMesh(core_axis_name="c", subcore_axis_name="s",
                                  num_cores=NC, num_subcores=NS)
    cp = pltpu.CompilerParams()
    if "needs_layout_passes" in pltpu.CompilerParams.__dataclass_fields__:
        cp = dataclasses.replace(cp, needs_layout_passes=False)
    cp = dataclasses.replace(cp, use_tc_tiling_on_sc=False)

    @functools.partial(
        pl.kernel,
        out_type=jax.ShapeDtypeStruct((NC, N_NODES, FEAT), jnp.float32),
        mesh=mesh,
        scratch_types=[
            pltpu.VMEM((K, FEAT // 2), jnp.int32),
            pltpu.VMEM((K, FEAT // 2), jnp.int32),
            pltpu.VMEM((4, K), jnp.int32),
            pltpu.VMEM((4, K), jnp.int32),
            pltpu.VMEM((K, FEAT), jnp.float32),
            pltpu.VMEM((K, FEAT), jnp.float32),
            pltpu.VMEM((K,), jnp.int32),
            pltpu.VMEM((K,), jnp.int32),
            pltpu.VMEM_SHARED((N_NODES, FEAT), jnp.float32),
            pltpu.SemaphoreType.DMA,
            pltpu.SemaphoreType.DMA,
            pltpu.SemaphoreType.DMA,
            pltpu.SemaphoreType.DMA,
            pltpu.SemaphoreType.DMA,
            pltpu.SemaphoreType.DMA,
        ],
        compiler_params=cp,
    )
    def k(h_hbm, meta_hbm, z_hbm, out_hbm,
          rows0, rows1, meta0, meta1, scaled0, scaled1, dstb0, dstb1, acc_s,
          gsem0, gsem1, msem0, msem1, ssem0, ssem1):
        cid = lax.axis_index("c")
        sid = lax.axis_index("s")
        wid = cid * NS + sid

        r0 = sid * RPT
        pltpu.sync_copy(z_hbm.at[pl.ds(r0, RPT)], acc_s.at[pl.ds(r0, RPT)])

        @pl.when(sid == NS - 1)
        def _():
            pltpu.sync_copy(z_hbm.at[pl.ds(NS * RPT, TAIL)],
                            acc_s.at[pl.ds(NS * RPT, TAIL)])

        plsc.subcore_barrier()

        pltpu.sync_copy(meta_hbm.at[wid, 0, pl.ds(0, 4)], meta0)
        pltpu.async_copy(h_hbm.at[meta0.at[0]], rows0, gsem0)
        pltpu.async_copy(meta_hbm.at[wid, 1, pl.ds(0, 4)], meta1, msem1)

        def chunk(kk, rows_c, meta_c, gsem_c, msem_c, scaled_c, dstb_c,
                  ssem_c, rows_n, meta_n, gsem_n, msem_n):
            @pl.when(kk + 1 < NCHUNK)
            def _():
                pltpu.make_async_copy(
                    meta_hbm.at[wid, kk + 1, pl.ds(0, 4)],
                    meta_n, msem_n).wait()
                pltpu.async_copy(h_hbm.at[meta_n.at[0]], rows_n, gsem_n)

            pltpu.make_async_copy(h_hbm.at[meta_c.at[0]], rows_c, gsem_c).wait()

            @pl.when(kk >= 2)
            def _():
                pltpu.make_async_copy(scaled_c, acc_s.at[dstb_c], ssem_c).wait()

            _scale_rows(rows_c, meta_c, scaled_c)

            for j in range(8):
                sl = pl.ds(j * 16, 16)
                dstb_c[sl] = meta_c[1, sl]
            pltpu.async_copy(scaled_c, acc_s.at[dstb_c], ssem_c, add=True)

            @pl.when(kk + 2 < NCHUNK)
            def _():
                pltpu.async_copy(meta_hbm.at[wid, kk + 2, pl.ds(0, 4)],
                                 meta_c, msem_c)

        @pl.loop(0, NCHUNK, step=2)
        def _(t):
            chunk(t, rows0, meta0, gsem0, msem0, scaled0, dstb0, ssem0,
                  rows1, meta1, gsem1, msem1)
            chunk(t + 1, rows1, meta1, gsem1, msem1, scaled1, dstb1, ssem1,
                  rows0, meta0, gsem0, msem0)

        pltpu.make_async_copy(scaled0, acc_s.at[dstb0], ssem0).wait()
        pltpu.make_async_copy(scaled1, acc_s.at[dstb1], ssem1).wait()

        plsc.subcore_barrier()
        pltpu.sync_copy(acc_s.at[pl.ds(r0, RPT)],
                        out_hbm.at[cid].at[pl.ds(r0, RPT)])

        @pl.when(sid == NS - 1)
        def _():
            pltpu.sync_copy(acc_s.at[pl.ds(NS * RPT, TAIL)],
                            out_hbm.at[cid].at[pl.ds(NS * RPT, TAIL)])

    return k(hp, meta, zeros)


def kernel(x, W, edge_index, edge_weight):
    pad = E_PAD - N_EDGES
    src = jnp.concatenate(
        [edge_index[0].astype(jnp.int32), jnp.zeros((pad,), jnp.int32)])
    dst = jnp.concatenate(
        [edge_index[1].astype(jnp.int32), jnp.zeros((pad,), jnp.int32)])
    ewb = jnp.concatenate(
        [edge_weight.astype(jnp.float32), jnp.zeros((pad,), jnp.float32)]
    ).view(jnp.int32)
    fill = jnp.zeros((NW, NCHUNK, 5, K), jnp.int32)
    meta = jnp.concatenate(
        [src.reshape(NW, NCHUNK, 1, K), dst.reshape(NW, NCHUNK, 1, K),
         ewb.reshape(NW, NCHUNK, 1, K), fill], axis=2)
    Wp = jnp.take(W, jnp.asarray(_PERM), axis=1)
    h16 = _matmul(x, Wp).astype(jnp.bfloat16)
    hp = lax.bitcast_convert_type(
        h16.reshape(N_NODES, FEAT // 2, 2), jnp.int32)
    zeros = jnp.zeros((N_NODES, FEAT), jnp.float32)
    parts = _sc_aggregate(hp, meta, zeros)
    return _combine(parts)


# speedup vs baseline: 1.0031x; 1.0028x over previous
import dataclasses
import functools

import jax
import jax.numpy as jnp
import numpy as np
from jax import lax
from jax.experimental import pallas as pl
from jax.experimental.pallas import tpu as pltpu
from jax.experimental.pallas import tpu_sc as plsc

N_NODES = 10000
FEAT = 128
N_EDGES = 320000

NC = 2
NS = 16
NW = NC * NS
K = 128
NCHUNK = 80
EPW = K * NCHUNK
E_PAD = NW * EPW
RPT = 624
TAIL = N_NODES - NS * RPT

_PERM = np.zeros(FEAT, np.int32)
for _q in range(FEAT // 32):
    for _c in range(32):
        _PERM[32 * _q + _c] = 32 * _q + (
            _c // 2 if _c % 2 == 0 else 16 + (_c - 1) // 2)


def _matmul(x, W):
    def body(x_ref, w_ref, o_ref):
        o_ref[...] = jax.lax.dot_general(
            x_ref[...], w_ref[...], (((1,), (0,)), ((), ())),
            preferred_element_type=jnp.float32,
            precision=jax.lax.Precision.HIGHEST)

    return pl.pallas_call(
        body,
        out_shape=jax.ShapeDtypeStruct((N_NODES, FEAT), jnp.float32),
    )(x, W)


def _combine(parts):
    def body(p_ref, o_ref):
        o_ref[...] = p_ref[0] + p_ref[1]

    return pl.pallas_call(
        body,
        out_shape=jax.ShapeDtypeStruct((N_NODES, FEAT), jnp.float32),
    )(parts)


def _scale_rows(rows_v, meta_v, scaled_v):
    @pl.loop(0, K, step=16)
    def _(g):
        wv = plsc.bitcast(meta_v[2, pl.ds(g, 16)], jnp.float32)
        dnums = lax.GatherDimensionNumbers(
            offset_dims=(), collapsed_slice_dims=(0,), start_index_map=(0,))
        for el in range(16):
            sp = lax.gather(wv, jnp.full((16, 1), el, jnp.int32), dnums, (1,),
                            mode=lax.GatherScatterMode.PROMISE_IN_BOUNDS)
            e = g + el
            for q in range(4):
                v = rows_v[e, pl.ds(q * 16, 16)]
                vb = plsc.bitcast(v, jnp.bfloat16)
                a, b = plsc.unpack(vb, format=plsc.PackFormat.INTERLEAVED)
                scaled_v[e, pl.ds(q * 32, 16)] = a * sp
                scaled_v[e, pl.ds(q * 32 + 16, 16)] = b * sp


def _sc_aggregate(hp, meta, zeros):
    mesh = plsc.VectorSubcoreMesh(core_axis_name="c", subcore_axis_name="s",
                                  num_cores=NC, num_subcores=NS)
    cp = pltpu.CompilerParams()
    if "needs_layout_passes" in pltpu.CompilerParams.__dataclass_fields__:
        cp = dataclasses.replace(cp, needs_layout_passes=False)
    cp = dataclasses.replace(cp, use_tc_tiling_on_sc=False)

    @functools.partial(
        pl.kernel,
        out_type=jax.ShapeDtypeStruct((NC, N_NODES, FEAT), jnp.float32),
        mesh=mesh,
        scratch_types=[
            pltpu.VMEM((K, FEAT // 2), jnp.int32),
            pltpu.VMEM((K, FEAT // 2), jnp.int32),
            pltpu.VMEM((4, K), jnp.int32),
            pltpu.VMEM((4, K), jnp.int32),
            pltpu.VMEM((K, FEAT), jnp.float32),
            pltpu.VMEM((K, FEAT), jnp.float32),
            pltpu.VMEM((K,), jnp.int32),
            pltpu.VMEM((K,), jnp.int32),
            pltpu.VMEM_SHARED((N_NODES, FEAT), jnp.float32),
            pltpu.SemaphoreType.DMA,
            pltpu.SemaphoreType.DMA,
            pltpu.SemaphoreType.DMA,
            pltpu.SemaphoreType.DMA,
            pltpu.SemaphoreType.DMA,
            pltpu.SemaphoreType.DMA,
        ],
        compiler_params=cp,
    )
    def k(h_hbm, meta_hbm, z_hbm, out_hbm,
          rows0, rows1, meta0, meta1, scaled0, scaled1, dstb0, dstb1, acc_s,
          gsem0, gsem1, msem0, msem1, ssem0, ssem1):
        cid = lax.axis_index("c")
        sid = lax.axis_index("s")
        wid = cid * NS + sid

        pltpu.sync_copy(meta_hbm.at[wid, 0, pl.ds(0, 4)], meta0)
        pltpu.async_copy(h_hbm.at[meta0.at[0]], rows0, gsem0)
        pltpu.async_copy(meta_hbm.at[wid, 1, pl.ds(0, 4)], meta1, msem1)

        r0 = sid * RPT
        pltpu.sync_copy(z_hbm.at[pl.ds(r0, RPT)], acc_s.at[pl.ds(r0, RPT)])

        @pl.when(sid == NS - 1)
        def _():
            pltpu.sync_copy(z_hbm.at[pl.ds(NS * RPT, TAIL)],
                            acc_s.at[pl.ds(NS * RPT, TAIL)])

        plsc.subcore_barrier()

        def chunk(kk, rows_c, meta_c, gsem_c, msem_c, scaled_c, dstb_c,
                  ssem_c, rows_n, meta_n, gsem_n, msem_n):
            @pl.when(kk + 1 < NCHUNK)
            def _():
                pltpu.make_async_copy(
                    meta_hbm.at[wid, kk + 1, pl.ds(0, 4)],
                    meta_n, msem_n).wait()
                pltpu.async_copy(h_hbm.at[meta_n.at[0]], rows_n, gsem_n)

            pltpu.make_async_copy(h_hbm.at[meta_c.at[0]], rows_c, gsem_c).wait()

            @pl.when(kk >= 2)
            def _():
                pltpu.make_async_copy(scaled_c, acc_s.at[dstb_c], ssem_c).wait()

            _scale_rows(rows_c, meta_c, scaled_c)

            for j in range(8):
                sl = pl.ds(j * 16, 16)
                dstb_c[sl] = meta_c[1, sl]
            pltpu.async_copy(scaled_c, acc_s.at[dstb_c], ssem_c, add=True)

            @pl.when(kk + 2 < NCHUNK)
            def _():
                pltpu.async_copy(meta_hbm.at[wid, kk + 2, pl.ds(0, 4)],
                                 meta_c, msem_c)

        @pl.loop(0, NCHUNK, step=2)
        def _(t):
            chunk(t, rows0, meta0, gsem0, msem0, scaled0, dstb0, ssem0,
                  rows1, meta1, gsem1, msem1)
            chunk(t + 1, rows1, meta1, gsem1, msem1, scaled1, dstb1, ssem1,
                  rows0, meta0, gsem0, msem0)

        pltpu.make_async_copy(scaled0, acc_s.at[dstb0], ssem0).wait()
        pltpu.make_async_copy(scaled1, acc_s.at[dstb1], ssem1).wait()

        plsc.subcore_barrier()
        pltpu.sync_copy(acc_s.at[pl.ds(r0, RPT)],
                        out_hbm.at[cid].at[pl.ds(r0, RPT)])

        @pl.when(sid == NS - 1)
        def _():
            pltpu.sync_copy(acc_s.at[pl.ds(NS * RPT, TAIL)],
                            out_hbm.at[cid].at[pl.ds(NS * RPT, TAIL)])

    return k(hp, meta, zeros)


def kernel(x, W, edge_index, edge_weight):
    pad = E_PAD - N_EDGES
    src = jnp.concatenate(
        [edge_index[0].astype(jnp.int32), jnp.zeros((pad,), jnp.int32)])
    dst = jnp.concatenate(
        [edge_index[1].astype(jnp.int32), jnp.zeros((pad,), jnp.int32)])
    ewb = jnp.concatenate(
        [edge_weight.astype(jnp.float32), jnp.zeros((pad,), jnp.float32)]
    ).view(jnp.int32)
    fill = jnp.zeros((NW, NCHUNK, 5, K), jnp.int32)
    meta = jnp.concatenate(
        [src.reshape(NW, NCHUNK, 1, K), dst.reshape(NW, NCHUNK, 1, K),
         ewb.reshape(NW, NCHUNK, 1, K), fill], axis=2)
    Wp = jnp.take(W, jnp.asarray(_PERM), axis=1)
    h16 = _matmul(x, Wp).astype(jnp.bfloat16)
    hp = lax.bitcast_convert_type(
        h16.reshape(N_NODES, FEAT // 2, 2), jnp.int32)
    zeros = jnp.zeros((N_NODES, FEAT), jnp.float32)
    parts = _sc_aggregate(hp, meta, zeros)
    return _combine(parts)
